# baseline (speedup 1.0000x reference)
import jax
import jax.numpy as jnp
from jax.experimental import pallas as pl

N = 10000
E = 320000
D = 128
H = 8
C = 16


def _tconv(h, src, dst, p, l):
    n = h.shape[0]
    q = (h @ p['l%d_Wq' % l] + p['l%d_bq' % l]).reshape(n, H, C)
    k = (h @ p['l%d_Wk' % l] + p['l%d_bk' % l]).reshape(n, H, C)
    v = (h @ p['l%d_Wv' % l] + p['l%d_bv' % l]).reshape(n, H, C)
    alpha = (q[dst] * k[src]).sum(-1) / jnp.sqrt(jnp.float32(C))
    amax = jax.ops.segment_max(alpha, dst, num_segments=n)
    amax = jnp.where(jnp.isfinite(amax), amax, 0.0)
    ex = jnp.exp(alpha - amax[dst])
    denom = jax.ops.segment_sum(ex, dst, num_segments=n)
    attn = ex / (denom[dst] + 1e-16)
    msg = v[src] * attn[:, :, None]
    out = jax.ops.segment_sum(msg, dst, num_segments=n).reshape(n, H * C)
    return out + h @ p['l%d_Ws' % l] + p['l%d_bs' % l]


def kernel(x, edge_index, params):
    src = edge_index[0]
    dst = edge_index[1]
    h = x
    for l in range(4):
        h = _tconv(h, src, dst, params, l)
        if l < 3:
            h = jax.nn.gelu(h, approximate=False)
    return h @ params['Wf'] + params['bf']


# speedup vs baseline: 28.5438x; 28.5438x over previous
import functools

import numpy as np
import jax
import jax.numpy as jnp
from jax import lax
from jax.experimental import pallas as pl
from jax.experimental.pallas import tpu as pltpu
from jax.experimental.pallas import tpu_sc as plsc

N = 10000
E = 320000
D = 128
H = 8
C = 16

BN = 1000
BE = 2000
CH = 80
NW = 32
EPW = E // NW
NCHW = EPW // CH
EPS = E // 16
NCHS = EPS // CH
NP = 10112
TPN = NP // 16

_INV_SQRT2 = np.float32(1.0 / np.sqrt(2.0))

_S2 = np.equal.outer(np.arange(D) // C, np.arange(D) // C).astype(np.float32)


def _proj0_body(h_ref, w_ref, b_ref, q_ref, k_ref, v_ref, s_ref):
    y = jnp.dot(h_ref[...], w_ref[...], preferred_element_type=jnp.float32)
    y = y + b_ref[...]
    q_ref[...] = y[:, 0:D] * 0.25
    k_ref[...] = y[:, D:2 * D]
    v_ref[...] = y[:, 2 * D:3 * D]
    s_ref[...] = y[:, 3 * D:4 * D]


def _merge_proj_body(acc_ref, den_ref, skip_ref, w_ref, b_ref,
                     q_ref, k_ref, v_ref, s_ref):
    hh = acc_ref[...] / (den_ref[...] + 1e-16) + skip_ref[...]
    hh = hh * 0.5 * (1.0 + lax.erf(hh * _INV_SQRT2))
    y = jnp.dot(hh, w_ref[...], preferred_element_type=jnp.float32)
    y = y + b_ref[...]
    q_ref[...] = y[:, 0:D] * 0.25
    k_ref[...] = y[:, D:2 * D]
    v_ref[...] = y[:, 2 * D:3 * D]
    s_ref[...] = y[:, 3 * D:4 * D]


def _final_body(acc_ref, den_ref, skip_ref, w_ref, b_ref, o_ref):
    hh = acc_ref[...] / (den_ref[...] + 1e-16) + skip_ref[...]
    o_ref[...] = jnp.dot(hh, w_ref[...],
                         preferred_element_type=jnp.float32) + b_ref[...]


def _emax_body(qg_ref, kg_ref, s2_ref, out_ref):
    i = pl.program_id(0)
    p = qg_ref[...] * kg_ref[...]
    ab = jnp.dot(p, s2_ref[...], preferred_element_type=jnp.float32)
    m = jnp.max(ab)[None, None]
    out_ref[...] = jnp.where(i == 0, m, jnp.maximum(out_ref[...], m))


def _e2_body(qg_ref, kg_ref, vg_ref, g_ref, s2_ref, msg_ref, exb_ref):
    p = qg_ref[...] * kg_ref[...]
    ab = jnp.dot(p, s2_ref[...], preferred_element_type=jnp.float32)
    ex = jnp.exp(ab - g_ref[...])
    msg_ref[...] = ex * vg_ref[...]
    exb_ref[...] = ex


def _proj0(h, wcat, bcat):
    return pl.pallas_call(
        _proj0_body,
        grid=(N // BN,),
        in_specs=[
            pl.BlockSpec((BN, D), lambda i: (i, 0)),
            pl.BlockSpec((D, 4 * D), lambda i: (0, 0)),
            pl.BlockSpec((1, 4 * D), lambda i: (0, 0)),
        ],
        out_specs=[pl.BlockSpec((BN, D), lambda i: (i, 0))] * 4,
        out_shape=[jax.ShapeDtypeStruct((N, D), jnp.float32)] * 4,
    )(h, wcat, bcat)


def _merge_proj(acc, den, skip, wcat, bcat):
    return pl.pallas_call(
        _merge_proj_body,
        grid=(N // BN,),
        in_specs=[
            pl.BlockSpec((BN, D), lambda i: (i, 0)),
            pl.BlockSpec((BN, D), lambda i: (i, 0)),
            pl.BlockSpec((BN, D), lambda i: (i, 0)),
            pl.BlockSpec((D, 4 * D), lambda i: (0, 0)),
            pl.BlockSpec((1, 4 * D), lambda i: (0, 0)),
        ],
        out_specs=[pl.BlockSpec((BN, D), lambda i: (i, 0))] * 4,
        out_shape=[jax.ShapeDtypeStruct((N, D), jnp.float32)] * 4,
    )(acc, den, skip, wcat, bcat)


def _final(acc, den, skip, wf, bf):
    return pl.pallas_call(
        _final_body,
        grid=(N // BN,),
        in_specs=[
            pl.BlockSpec((BN, D), lambda i: (i, 0)),
            pl.BlockSpec((BN, D), lambda i: (i, 0)),
            pl.BlockSpec((BN, D), lambda i: (i, 0)),
            pl.BlockSpec((D, D), lambda i: (0, 0)),
            pl.BlockSpec((1, D), lambda i: (0, 0)),
        ],
        out_specs=pl.BlockSpec((BN, D), lambda i: (i, 0)),
        out_shape=jax.ShapeDtypeStruct((N, D), jnp.float32),
    )(acc, den, skip, wf, bf)


def _emax(qg, kg, s2):
    return pl.pallas_call(
        _emax_body,
        grid=(E // BE,),
        in_specs=[
            pl.BlockSpec((BE, D), lambda i: (i, 0)),
            pl.BlockSpec((BE, D), lambda i: (i, 0)),
            pl.BlockSpec((D, D), lambda i: (0, 0)),
        ],
        out_specs=pl.BlockSpec((1, 1), lambda i: (0, 0)),
        out_shape=jax.ShapeDtypeStruct((1, 1), jnp.float32),
    )(qg, kg, s2)


def _e2(qg, kg, vg, g, s2):
    return pl.pallas_call(
        _e2_body,
        grid=(E // BE,),
        in_specs=[
            pl.BlockSpec((BE, D), lambda i: (i, 0)),
            pl.BlockSpec((BE, D), lambda i: (i, 0)),
            pl.BlockSpec((BE, D), lambda i: (i, 0)),
            pl.BlockSpec((1, 1), lambda i: (0, 0)),
            pl.BlockSpec((D, D), lambda i: (0, 0)),
        ],
        out_specs=[
            pl.BlockSpec((BE, D), lambda i: (i, 0)),
            pl.BlockSpec((BE, D), lambda i: (i, 0)),
        ],
        out_shape=[
            jax.ShapeDtypeStruct((E, D), jnp.float32),
            jax.ShapeDtypeStruct((E, D), jnp.float32),
        ],
    )(qg, kg, vg, g, s2)


_MESH = plsc.VectorSubcoreMesh(core_axis_name="c", subcore_axis_name="s")


@functools.partial(
    pl.kernel,
    mesh=_MESH,
    out_type=[jax.ShapeDtypeStruct((E, D), jnp.float32)] * 3,
    scratch_types=[
        pltpu.VMEM((CH,), jnp.int32),
        pltpu.VMEM((CH,), jnp.int32),
        pltpu.VMEM((CH, D), jnp.float32),
        pltpu.VMEM((CH, D), jnp.float32),
        pltpu.VMEM((CH, D), jnp.float32),
        pltpu.SemaphoreType.DMA,
    ],
)
def _sc_gather(src_hbm, dst_hbm, q_hbm, k_hbm, v_hbm,
               qg_hbm, kg_hbm, vg_hbm,
               didx, sidx, qb, kb, vb, sem):
    wid = lax.axis_index("s") * 2 + lax.axis_index("c")

    def body(j, carry):
        base = wid * EPW + j * CH
        pltpu.sync_copy(dst_hbm.at[pl.ds(base, CH)], didx)
        pltpu.sync_copy(src_hbm.at[pl.ds(base, CH)], sidx)
        cq = pltpu.async_copy(q_hbm.at[didx], qb, sem)
        ck = pltpu.async_copy(k_hbm.at[sidx], kb, sem)
        cv = pltpu.async_copy(v_hbm.at[sidx], vb, sem)
        cq.wait()
        ck.wait()
        cv.wait()
        pltpu.sync_copy(qb, qg_hbm.at[pl.ds(base, CH)])
        pltpu.sync_copy(kb, kg_hbm.at[pl.ds(base, CH)])
        pltpu.sync_copy(vb, vg_hbm.at[pl.ds(base, CH)])
        return carry

    lax.fori_loop(0, NCHW, body, 0)


@functools.partial(
    pl.kernel,
    mesh=_MESH,
    out_type=[
        jax.ShapeDtypeStruct((NP, D), jnp.float32),
        jax.ShapeDtypeStruct((NP, D), jnp.float32),
    ],
    scratch_types=[
        pltpu.VMEM((CH,), jnp.int32),
        pltpu.VMEM((CH, D), jnp.float32),
        pltpu.VMEM((8, D), jnp.float32),
        pltpu.VMEM_SHARED((NP, D), jnp.float32),
        pltpu.SemaphoreType.DMA,
    ],
)
def _sc_scatter(dst_hbm, msg_hbm, exb_hbm, acc_hbm, den_hbm,
                didx, mb, zb, acc_s, sem):
    cid = lax.axis_index("c")
    sid = lax.axis_index("s")

    zero = jnp.zeros((16,), jnp.float32)
    for r in range(8):
        for h in range(D // 16):
            zb[r, pl.ds(h * 16, 16)] = zero
    row0 = sid * TPN

    def zbody(z, carry):
        pltpu.sync_copy(zb, acc_s.at[pl.ds(row0 + z * 8, 8)])
        return carry

    lax.fori_loop(0, TPN // 8, zbody, 0)
    plsc.subcore_barrier()

    def mbody(j, carry):
        base = sid * EPS + j * CH
        pltpu.sync_copy(dst_hbm.at[pl.ds(base, CH)], didx)
        pltpu.sync_copy(msg_hbm.at[pl.ds(base, CH)], mb)
        pltpu.sync_copy(mb, acc_s.at[didx], add=True)
        return carry

    def dbody(j, carry):
        base = sid * EPS + j * CH
        pltpu.sync_copy(dst_hbm.at[pl.ds(base, CH)], didx)
        pltpu.sync_copy(exb_hbm.at[pl.ds(base, CH)], mb)
        pltpu.sync_copy(mb, acc_s.at[didx], add=True)
        return carry

    @pl.when(cid == 0)
    def _():
        lax.fori_loop(0, NCHS, mbody, 0)

    @pl.when(cid == 1)
    def _():
        lax.fori_loop(0, NCHS, dbody, 0)

    plsc.subcore_barrier()

    @pl.when(cid == 0)
    def _():
        pltpu.sync_copy(acc_s.at[pl.ds(row0, TPN)],
                        acc_hbm.at[pl.ds(row0, TPN)])

    @pl.when(cid == 1)
    def _():
        pltpu.sync_copy(acc_s.at[pl.ds(row0, TPN)],
                        den_hbm.at[pl.ds(row0, TPN)])


def kernel(x, edge_index, params):
    src = edge_index[0]
    dst = edge_index[1]
    s2 = jnp.asarray(_S2)

    h = x
    skip = None
    acc = den = None
    for l in range(4):
        wcat = jnp.concatenate(
            [params['l%d_W%s' % (l, nm)] for nm in 'qkvs'], axis=1)
        bcat = jnp.concatenate(
            [params['l%d_b%s' % (l, nm)] for nm in 'qkvs'])[None, :]
        if l == 0:
            q, k, v, s = _proj0(h, wcat, bcat)
        else:
            q, k, v, s = _merge_proj(acc, den, skip, wcat, bcat)
        qg, kg, vg = _sc_gather(src, dst, q, k, v)
        g = _emax(qg, kg, s2)
        msg, exb = _e2(qg, kg, vg, g, s2)
        acc, den = _sc_scatter(dst, msg, exb)
        skip = s
    return _final(acc, den, skip, params['Wf'], params['bf'][None, :])


# speedup vs baseline: 53.0866x; 1.8598x over previous
import functools

import numpy as np
import jax
import jax.numpy as jnp
from jax import lax
from jax.experimental import pallas as pl
from jax.experimental.pallas import tpu as pltpu
from jax.experimental.pallas import tpu_sc as plsc

N = 10000
E = 320000
D = 128
H = 8
C = 16

BN = 1000
BE = 2000
CH = 80
NW = 32
EPW = E // NW
NCHW = EPW // CH
EPS = E // 16
NCHS = EPS // CH
NP = 10112
TPN = NP // 16

_INV_SQRT2 = np.float32(1.0 / np.sqrt(2.0))

_S2 = np.equal.outer(np.arange(D) // C, np.arange(D) // C).astype(np.float32)


def _head_sq_max(z, s2):
    zn = jnp.dot(z * z, s2, preferred_element_type=jnp.float32)
    return jnp.max(zn, axis=0, keepdims=True)


def _proj_tail(y, s2, i, q_ref, k_ref, v_ref, s_ref, qm_ref, km_ref):
    q = y[:, 0:D] * 0.25
    k = y[:, D:2 * D]
    q_ref[...] = q
    k_ref[...] = k
    v_ref[...] = y[:, 2 * D:3 * D]
    s_ref[...] = y[:, 3 * D:4 * D]
    qm = _head_sq_max(q, s2)
    km = _head_sq_max(k, s2)
    qm_ref[...] = jnp.where(i == 0, qm, jnp.maximum(qm_ref[...], qm))
    km_ref[...] = jnp.where(i == 0, km, jnp.maximum(km_ref[...], km))


def _proj0_body(h_ref, w_ref, b_ref, s2_ref,
                q_ref, k_ref, v_ref, s_ref, qm_ref, km_ref):
    i = pl.program_id(0)
    y = jnp.dot(h_ref[...], w_ref[...], preferred_element_type=jnp.float32)
    y = y + b_ref[...]
    _proj_tail(y, s2_ref[...], i, q_ref, k_ref, v_ref, s_ref, qm_ref, km_ref)


def _merge_proj_body(acc_ref, den_ref, skip_ref, w_ref, b_ref, s2_ref,
                     q_ref, k_ref, v_ref, s_ref, qm_ref, km_ref):
    i = pl.program_id(0)
    hh = acc_ref[...] / (den_ref[...] + 1e-16) + skip_ref[...]
    hh = hh * 0.5 * (1.0 + lax.erf(hh * _INV_SQRT2))
    y = jnp.dot(hh, w_ref[...], preferred_element_type=jnp.float32)
    y = y + b_ref[...]
    _proj_tail(y, s2_ref[...], i, q_ref, k_ref, v_ref, s_ref, qm_ref, km_ref)


def _final_body(acc_ref, den_ref, skip_ref, w_ref, b_ref, o_ref):
    hh = acc_ref[...] / (den_ref[...] + 1e-16) + skip_ref[...]
    o_ref[...] = jnp.dot(hh, w_ref[...],
                         preferred_element_type=jnp.float32) + b_ref[...]


def _e2_body(qg_ref, kg_ref, vg_ref, qm_ref, km_ref, s2_ref,
             msg_ref, exb_ref):
    g = jnp.sqrt(qm_ref[...] * km_ref[...])
    p = qg_ref[...] * kg_ref[...]
    ab = jnp.dot(p, s2_ref[...], preferred_element_type=jnp.float32)
    ex = jnp.exp(ab - g)
    msg_ref[...] = ex * vg_ref[...]
    exb_ref[...] = ex


_PROJ_OUTS = (
    [pl.BlockSpec((BN, D), lambda i: (i, 0))] * 4
    + [pl.BlockSpec((1, D), lambda i: (0, 0))] * 2,
    [jax.ShapeDtypeStruct((N, D), jnp.float32)] * 4
    + [jax.ShapeDtypeStruct((1, D), jnp.float32)] * 2,
)


def _proj0(h, wcat, bcat, s2):
    return pl.pallas_call(
        _proj0_body,
        grid=(N // BN,),
        in_specs=[
            pl.BlockSpec((BN, D), lambda i: (i, 0)),
            pl.BlockSpec((D, 4 * D), lambda i: (0, 0)),
            pl.BlockSpec((1, 4 * D), lambda i: (0, 0)),
            pl.BlockSpec((D, D), lambda i: (0, 0)),
        ],
        out_specs=_PROJ_OUTS[0],
        out_shape=_PROJ_OUTS[1],
    )(h, wcat, bcat, s2)


def _merge_proj(acc, den, skip, wcat, bcat, s2):
    return pl.pallas_call(
        _merge_proj_body,
        grid=(N // BN,),
        in_specs=[
            pl.BlockSpec((BN, D), lambda i: (i, 0)),
            pl.BlockSpec((BN, D), lambda i: (i, 0)),
            pl.BlockSpec((BN, D), lambda i: (i, 0)),
            pl.BlockSpec((D, 4 * D), lambda i: (0, 0)),
            pl.BlockSpec((1, 4 * D), lambda i: (0, 0)),
            pl.BlockSpec((D, D), lambda i: (0, 0)),
        ],
        out_specs=_PROJ_OUTS[0],
        out_shape=_PROJ_OUTS[1],
    )(acc, den, skip, wcat, bcat, s2)


def _final(acc, den, skip, wf, bf):
    return pl.pallas_call(
        _final_body,
        grid=(N // BN,),
        in_specs=[
            pl.BlockSpec((BN, D), lambda i: (i, 0)),
            pl.BlockSpec((BN, D), lambda i: (i, 0)),
            pl.BlockSpec((BN, D), lambda i: (i, 0)),
            pl.BlockSpec((D, D), lambda i: (0, 0)),
            pl.BlockSpec((1, D), lambda i: (0, 0)),
        ],
        out_specs=pl.BlockSpec((BN, D), lambda i: (i, 0)),
        out_shape=jax.ShapeDtypeStruct((N, D), jnp.float32),
    )(acc, den, skip, wf, bf)


def _e2(qg, kg, vg, qm, km, s2):
    return pl.pallas_call(
        _e2_body,
        grid=(E // BE,),
        in_specs=[
            pl.BlockSpec((BE, D), lambda i: (i, 0)),
            pl.BlockSpec((BE, D), lambda i: (i, 0)),
            pl.BlockSpec((BE, D), lambda i: (i, 0)),
            pl.BlockSpec((1, D), lambda i: (0, 0)),
            pl.BlockSpec((1, D), lambda i: (0, 0)),
            pl.BlockSpec((D, D), lambda i: (0, 0)),
        ],
        out_specs=[
            pl.BlockSpec((BE, D), lambda i: (i, 0)),
            pl.BlockSpec((BE, D), lambda i: (i, 0)),
        ],
        out_shape=[
            jax.ShapeDtypeStruct((E, D), jnp.float32),
            jax.ShapeDtypeStruct((E, D), jnp.float32),
        ],
    )(qg, kg, vg, qm, km, s2)


_MESH = plsc.VectorSubcoreMesh(core_axis_name="c", subcore_axis_name="s")


@functools.partial(
    pl.kernel,
    mesh=_MESH,
    out_type=[jax.ShapeDtypeStruct((E, D), jnp.float32)] * 3,
    scratch_types=[
        pltpu.VMEM((EPW,), jnp.int32),
        pltpu.VMEM((EPW,), jnp.int32),
    ]
    + [pltpu.VMEM((CH, D), jnp.float32)] * 9
    + [pltpu.SemaphoreType.DMA] * 6,
)
def _sc_gather(src_hbm, dst_hbm, q_hbm, k_hbm, v_hbm,
               qg_hbm, kg_hbm, vg_hbm,
               dxa, sxa, qb0, qb1, qb2, kb0, kb1, kb2, vb0, vb1, vb2,
               gs0, gs1, gs2, ss0, ss1, ss2):
    wid = lax.axis_index("s") * 2 + lax.axis_index("c")
    ebase = wid * EPW
    pltpu.sync_copy(dst_hbm.at[pl.ds(ebase, EPW)], dxa)
    pltpu.sync_copy(src_hbm.at[pl.ds(ebase, EPW)], sxa)

    qbs = (qb0, qb1, qb2)
    kbs = (kb0, kb1, kb2)
    vbs = (vb0, vb1, vb2)
    gsems = (gs0, gs1, gs2)
    ssems = (ss0, ss1, ss2)

    def fire_gathers(j, b):
        di = dxa.at[pl.ds(j * CH, CH)]
        si = sxa.at[pl.ds(j * CH, CH)]
        pltpu.async_copy(q_hbm.at[di], qbs[b], gsems[b])
        pltpu.async_copy(k_hbm.at[si], kbs[b], gsems[b])
        pltpu.async_copy(v_hbm.at[si], vbs[b], gsems[b])

    def wait_gathers(b):
        pltpu.make_async_copy(q_hbm.at[pl.ds(0, CH)], qbs[b], gsems[b]).wait()
        pltpu.make_async_copy(k_hbm.at[pl.ds(0, CH)], kbs[b], gsems[b]).wait()
        pltpu.make_async_copy(v_hbm.at[pl.ds(0, CH)], vbs[b], gsems[b]).wait()

    def fire_stores(j, b):
        base = ebase + j * CH
        pltpu.async_copy(qbs[b], qg_hbm.at[pl.ds(base, CH)], ssems[b])
        pltpu.async_copy(kbs[b], kg_hbm.at[pl.ds(base, CH)], ssems[b])
        pltpu.async_copy(vbs[b], vg_hbm.at[pl.ds(base, CH)], ssems[b])

    def wait_stores(b):
        pltpu.make_async_copy(qbs[b], qg_hbm.at[pl.ds(0, CH)], ssems[b]).wait()
        pltpu.make_async_copy(kbs[b], kg_hbm.at[pl.ds(0, CH)], ssems[b]).wait()
        pltpu.make_async_copy(vbs[b], vg_hbm.at[pl.ds(0, CH)], ssems[b]).wait()

    fire_gathers(0, 0)
    fire_gathers(1, 1)

    def body(t, carry):
        for i in range(3):
            j = 3 * t + i

            @pl.when(j < NCHW)
            def _():
                wait_gathers(i)
                fire_stores(j, i)
                jn = j + 2
                bn = (i + 2) % 3

                @pl.when(jn < NCHW)
                def _():
                    @pl.when(j >= 1)
                    def _():
                        wait_stores(bn)

                    fire_gathers(jn, bn)

        return carry

    lax.fori_loop(0, (NCHW + 2) // 3, body, 0)
    wait_stores(2)
    wait_stores(0)
    wait_stores(1)


@functools.partial(
    pl.kernel,
    mesh=_MESH,
    out_type=[
        jax.ShapeDtypeStruct((NP, D), jnp.float32),
        jax.ShapeDtypeStruct((NP, D), jnp.float32),
    ],
    scratch_types=[
        pltpu.VMEM((CH,), jnp.int32),
        pltpu.VMEM((CH,), jnp.int32),
        pltpu.VMEM((CH,), jnp.int32),
    ]
    + [pltpu.VMEM((CH, D), jnp.float32)] * 3
    + [
        pltpu.VMEM((8, D), jnp.float32),
        pltpu.VMEM_SHARED((NP, D), jnp.float32),
    ]
    + [pltpu.SemaphoreType.DMA] * 3,
)
def _sc_scatter(dst_hbm, msg_hbm, exb_hbm, acc_hbm, den_hbm,
                di0, di1, di2, mb0, mb1, mb2, zb, acc_s, ls0, ls1, ls2):
    cid = lax.axis_index("c")
    sid = lax.axis_index("s")

    zero = jnp.zeros((16,), jnp.float32)
    for r in range(8):
        for h in range(D // 16):
            zb[r, pl.ds(h * 16, 16)] = zero
    row0 = sid * TPN

    def zbody(z, carry):
        pltpu.sync_copy(zb, acc_s.at[pl.ds(row0 + z * 8, 8)])
        return carry

    lax.fori_loop(0, TPN // 8, zbody, 0)
    plsc.subcore_barrier()

    dis = (di0, di1, di2)
    mbs = (mb0, mb1, mb2)
    lsems = (ls0, ls1, ls2)
    ebase = sid * EPS

    def fire_loads(src_ref, j, b):
        base = ebase + j * CH
        pltpu.async_copy(dst_hbm.at[pl.ds(base, CH)], dis[b], lsems[b])
        pltpu.async_copy(src_ref.at[pl.ds(base, CH)], mbs[b], lsems[b])

    def wait_loads(b):
        pltpu.make_async_copy(dst_hbm.at[pl.ds(0, CH)], dis[b],
                              lsems[b]).wait()
        pltpu.make_async_copy(msg_hbm.at[pl.ds(0, CH)], mbs[b],
                              lsems[b]).wait()

    def run(src_ref):
        fire_loads(src_ref, 0, 0)
        fire_loads(src_ref, 1, 1)
        fire_loads(src_ref, 2, 2)

        def body(t, carry):
            for i in range(3):
                j = 3 * t + i

                @pl.when(j < NCHS)
                def _():
                    wait_loads(i)
                    pltpu.sync_copy(mbs[i], acc_s.at[dis[i]], add=True)

                    @pl.when(j + 3 < NCHS)
                    def _():
                        fire_loads(src_ref, j + 3, i)

            return carry

        lax.fori_loop(0, (NCHS + 2) // 3, body, 0)

    @pl.when(cid == 0)
    def _():
        run(msg_hbm)

    @pl.when(cid == 1)
    def _():
        run(exb_hbm)

    plsc.subcore_barrier()

    @pl.when(cid == 0)
    def _():
        pltpu.sync_copy(acc_s.at[pl.ds(row0, TPN)],
                        acc_hbm.at[pl.ds(row0, TPN)])

    @pl.when(cid == 1)
    def _():
        pltpu.sync_copy(acc_s.at[pl.ds(row0, TPN)],
                        den_hbm.at[pl.ds(row0, TPN)])


def kernel(x, edge_index, params):
    src = edge_index[0]
    dst = edge_index[1]
    s2 = jnp.asarray(_S2)

    h = x
    skip = None
    acc = den = None
    for l in range(4):
        wcat = jnp.concatenate(
            [params['l%d_W%s' % (l, nm)] for nm in 'qkvs'], axis=1)
        bcat = jnp.concatenate(
            [params['l%d_b%s' % (l, nm)] for nm in 'qkvs'])[None, :]
        if l == 0:
            q, k, v, s, qm, km = _proj0(h, wcat, bcat, s2)
        else:
            q, k, v, s, qm, km = _merge_proj(acc, den, skip, wcat, bcat, s2)
        qg, kg, vg = _sc_gather(src, dst, q, k, v)
        msg, exb = _e2(qg, kg, vg, qm, km, s2)
        acc, den = _sc_scatter(dst, msg, exb)
        skip = s
    return _final(acc, den, skip, params['Wf'], params['bf'][None, :])


# speedup vs baseline: 60.5314x; 1.1402x over previous
import functools

import numpy as np
import jax
import jax.numpy as jnp
from jax import lax
from jax.experimental import pallas as pl
from jax.experimental.pallas import tpu as pltpu
from jax.experimental.pallas import tpu_sc as plsc

N = 10000
E = 320000
D = 128
H = 8
C = 16

BN = 1000
BE = 2000
CH = 80
NW = 32
EPW = E // NW
NCHW = EPW // CH
EPS = E // 16
NCHS = EPS // CH
NP = 10112
TPN = NP // 16

_INV_SQRT2 = np.float32(1.0 / np.sqrt(2.0))

_S2 = np.equal.outer(np.arange(D) // C, np.arange(D) // C).astype(np.float32)


def _head_sq_max(z, s2):
    zn = jnp.dot(z * z, s2, preferred_element_type=jnp.float32)
    return jnp.max(zn, axis=0, keepdims=True)


def _proj_tail(y, s2, i, q_ref, k_ref, v_ref, s_ref, g_ref, qm_s, km_s):
    q = y[:, 0:D] * 0.25
    k = y[:, D:2 * D]
    q_ref[...] = q
    k_ref[...] = k
    v_ref[...] = y[:, 2 * D:3 * D]
    s_ref[...] = y[:, 3 * D:4 * D]
    qm = _head_sq_max(q, s2)
    km = _head_sq_max(k, s2)
    qm_s[...] = jnp.where(i == 0, qm, jnp.maximum(qm_s[...], qm))
    km_s[...] = jnp.where(i == 0, km, jnp.maximum(km_s[...], km))
    g_ref[...] = jnp.sqrt(qm_s[...] * km_s[...])


def _proj0_body(h_ref, w_ref, b_ref, s2_ref,
                q_ref, k_ref, v_ref, s_ref, g_ref, qm_s, km_s):
    i = pl.program_id(0)
    y = jnp.dot(h_ref[...], w_ref[...], preferred_element_type=jnp.float32)
    y = y + b_ref[...]
    _proj_tail(y, s2_ref[...], i, q_ref, k_ref, v_ref, s_ref, g_ref,
               qm_s, km_s)


def _merge_proj_body(acc_ref, den_ref, skip_ref, w_ref, b_ref, s2_ref,
                     q_ref, k_ref, v_ref, s_ref, g_ref, qm_s, km_s):
    i = pl.program_id(0)
    hh = acc_ref[...] / (den_ref[...] + 1e-16) + skip_ref[...]
    hh = hh * 0.5 * (1.0 + lax.erf(hh * _INV_SQRT2))
    y = jnp.dot(hh, w_ref[...], preferred_element_type=jnp.float32)
    y = y + b_ref[...]
    _proj_tail(y, s2_ref[...], i, q_ref, k_ref, v_ref, s_ref, g_ref,
               qm_s, km_s)


def _final_body(acc_ref, den_ref, skip_ref, w_ref, b_ref, o_ref):
    hh = acc_ref[...] / (den_ref[...] + 1e-16) + skip_ref[...]
    o_ref[...] = jnp.dot(hh, w_ref[...],
                         preferred_element_type=jnp.float32) + b_ref[...]


def _e2_body(pg_ref, vg_ref, g_ref, s2_ref, msg_ref, exb_ref):
    ab = jnp.dot(pg_ref[...], s2_ref[...],
                 preferred_element_type=jnp.float32)
    ex = jnp.exp(ab - g_ref[...])
    msg_ref[...] = ex * vg_ref[...]
    exb_ref[...] = ex


_PROJ_OUTS = (
    [pl.BlockSpec((BN, D), lambda i: (i, 0))] * 4
    + [pl.BlockSpec((1, D), lambda i: (0, 0))],
    [jax.ShapeDtypeStruct((N, D), jnp.float32)] * 4
    + [jax.ShapeDtypeStruct((1, D), jnp.float32)],
)
_PROJ_SCRATCH = [pltpu.VMEM((1, D), jnp.float32)] * 2


def _proj0(h, wcat, bcat, s2):
    return pl.pallas_call(
        _proj0_body,
        grid=(N // BN,),
        in_specs=[
            pl.BlockSpec((BN, D), lambda i: (i, 0)),
            pl.BlockSpec((D, 4 * D), lambda i: (0, 0)),
            pl.BlockSpec((1, 4 * D), lambda i: (0, 0)),
            pl.BlockSpec((D, D), lambda i: (0, 0)),
        ],
        out_specs=_PROJ_OUTS[0],
        out_shape=_PROJ_OUTS[1],
        scratch_shapes=_PROJ_SCRATCH,
    )(h, wcat, bcat, s2)


def _merge_proj(acc, den, skip, wcat, bcat, s2):
    return pl.pallas_call(
        _merge_proj_body,
        grid=(N // BN,),
        in_specs=[
            pl.BlockSpec((BN, D), lambda i: (i, 0)),
            pl.BlockSpec((BN, D), lambda i: (i, 0)),
            pl.BlockSpec((BN, D), lambda i: (i, 0)),
            pl.BlockSpec((D, 4 * D), lambda i: (0, 0)),
            pl.BlockSpec((1, 4 * D), lambda i: (0, 0)),
            pl.BlockSpec((D, D), lambda i: (0, 0)),
        ],
        out_specs=_PROJ_OUTS[0],
        out_shape=_PROJ_OUTS[1],
        scratch_shapes=_PROJ_SCRATCH,
    )(acc, den, skip, wcat, bcat, s2)


def _final(acc, den, skip, wf, bf):
    return pl.pallas_call(
        _final_body,
        grid=(N // BN,),
        in_specs=[
            pl.BlockSpec((BN, D), lambda i: (i, 0)),
            pl.BlockSpec((BN, D), lambda i: (i, 0)),
            pl.BlockSpec((BN, D), lambda i: (i, 0)),
            pl.BlockSpec((D, D), lambda i: (0, 0)),
            pl.BlockSpec((1, D), lambda i: (0, 0)),
        ],
        out_specs=pl.BlockSpec((BN, D), lambda i: (i, 0)),
        out_shape=jax.ShapeDtypeStruct((N, D), jnp.float32),
    )(acc, den, skip, wf, bf)


def _e2(pg, vg, g, s2):
    return pl.pallas_call(
        _e2_body,
        grid=(E // BE,),
        in_specs=[
            pl.BlockSpec((BE, D), lambda i: (i, 0)),
            pl.BlockSpec((BE, D), lambda i: (i, 0)),
            pl.BlockSpec((1, D), lambda i: (0, 0)),
            pl.BlockSpec((D, D), lambda i: (0, 0)),
        ],
        out_specs=[
            pl.BlockSpec((BE, D), lambda i: (i, 0)),
            pl.BlockSpec((BE, D), lambda i: (i, 0)),
        ],
        out_shape=[
            jax.ShapeDtypeStruct((E, D), jnp.float32),
            jax.ShapeDtypeStruct((E, D), jnp.float32),
        ],
    )(pg, vg, g, s2)


_MESH = plsc.VectorSubcoreMesh(core_axis_name="c", subcore_axis_name="s")


@functools.partial(
    pl.kernel,
    mesh=_MESH,
    out_type=[jax.ShapeDtypeStruct((E, D), jnp.float32)] * 2,
    scratch_types=[
        pltpu.VMEM((EPW,), jnp.int32),
        pltpu.VMEM((EPW,), jnp.int32),
    ]
    + [pltpu.VMEM((CH, D), jnp.float32)] * 9
    + [pltpu.SemaphoreType.DMA] * 6,
)
def _sc_gather(src_hbm, dst_hbm, q_hbm, k_hbm, v_hbm,
               pg_hbm, vg_hbm,
               dxa, sxa, qb0, qb1, qb2, kb0, kb1, kb2, vb0, vb1, vb2,
               gs0, gs1, gs2, ss0, ss1, ss2):
    wid = lax.axis_index("s") * 2 + lax.axis_index("c")
    ebase = wid * EPW
    pltpu.sync_copy(dst_hbm.at[pl.ds(ebase, EPW)], dxa)
    pltpu.sync_copy(src_hbm.at[pl.ds(ebase, EPW)], sxa)

    qbs = (qb0, qb1, qb2)
    kbs = (kb0, kb1, kb2)
    vbs = (vb0, vb1, vb2)
    gsems = (gs0, gs1, gs2)
    ssems = (ss0, ss1, ss2)

    def fire_gathers(j, b):
        di = dxa.at[pl.ds(j * CH, CH)]
        si = sxa.at[pl.ds(j * CH, CH)]
        pltpu.async_copy(q_hbm.at[di], qbs[b], gsems[b])
        pltpu.async_copy(k_hbm.at[si], kbs[b], gsems[b])
        pltpu.async_copy(v_hbm.at[si], vbs[b], gsems[b])

    def wait_gathers(b):
        pltpu.make_async_copy(q_hbm.at[pl.ds(0, CH)], qbs[b], gsems[b]).wait()
        pltpu.make_async_copy(k_hbm.at[pl.ds(0, CH)], kbs[b], gsems[b]).wait()
        pltpu.make_async_copy(v_hbm.at[pl.ds(0, CH)], vbs[b], gsems[b]).wait()

    def compute(b):
        qb, kb = qbs[b], kbs[b]

        def edge_body(t, carry):
            for u in range(4):
                e = 4 * t + u
                for h in range(D // C):
                    sl = pl.ds(h * C, C)
                    kb[e, sl] = qb[e, sl] * kb[e, sl]
            return carry

        lax.fori_loop(0, CH // 4, edge_body, 0)

    def fire_stores(j, b):
        base = ebase + j * CH
        pltpu.async_copy(kbs[b], pg_hbm.at[pl.ds(base, CH)], ssems[b])
        pltpu.async_copy(vbs[b], vg_hbm.at[pl.ds(base, CH)], ssems[b])

    def wait_stores(b):
        pltpu.make_async_copy(kbs[b], pg_hbm.at[pl.ds(0, CH)], ssems[b]).wait()
        pltpu.make_async_copy(vbs[b], vg_hbm.at[pl.ds(0, CH)], ssems[b]).wait()

    fire_gathers(0, 0)
    fire_gathers(1, 1)

    def body(t, carry):
        for i in range(3):
            j = 3 * t + i

            @pl.when(j < NCHW)
            def _():
                wait_gathers(i)
                compute(i)
                fire_stores(j, i)
                jn = j + 2
                bn = (i + 2) % 3

                @pl.when(jn < NCHW)
                def _():
                    @pl.when(j >= 1)
                    def _():
                        wait_stores(bn)

                    fire_gathers(jn, bn)

        return carry

    lax.fori_loop(0, (NCHW + 2) // 3, body, 0)
    wait_stores(2)
    wait_stores(0)
    wait_stores(1)


@functools.partial(
    pl.kernel,
    mesh=_MESH,
    out_type=[
        jax.ShapeDtypeStruct((NP, D), jnp.float32),
        jax.ShapeDtypeStruct((NP, D), jnp.float32),
    ],
    scratch_types=[
        pltpu.VMEM((CH,), jnp.int32),
        pltpu.VMEM((CH,), jnp.int32),
        pltpu.VMEM((CH,), jnp.int32),
    ]
    + [pltpu.VMEM((CH, D), jnp.float32)] * 3
    + [
        pltpu.VMEM((8, D), jnp.float32),
        pltpu.VMEM_SHARED((NP, D), jnp.float32),
    ]
    + [pltpu.SemaphoreType.DMA] * 3,
)
def _sc_scatter(dst_hbm, msg_hbm, exb_hbm, acc_hbm, den_hbm,
                di0, di1, di2, mb0, mb1, mb2, zb, acc_s, ls0, ls1, ls2):
    cid = lax.axis_index("c")
    sid = lax.axis_index("s")

    zero = jnp.zeros((16,), jnp.float32)
    for r in range(8):
        for h in range(D // 16):
            zb[r, pl.ds(h * 16, 16)] = zero
    row0 = sid * TPN

    def zbody(z, carry):
        pltpu.sync_copy(zb, acc_s.at[pl.ds(row0 + z * 8, 8)])
        return carry

    lax.fori_loop(0, TPN // 8, zbody, 0)
    plsc.subcore_barrier()

    dis = (di0, di1, di2)
    mbs = (mb0, mb1, mb2)
    lsems = (ls0, ls1, ls2)
    ebase = sid * EPS

    def fire_loads(src_ref, j, b):
        base = ebase + j * CH
        pltpu.async_copy(dst_hbm.at[pl.ds(base, CH)], dis[b], lsems[b])
        pltpu.async_copy(src_ref.at[pl.ds(base, CH)], mbs[b], lsems[b])

    def wait_loads(b):
        pltpu.make_async_copy(dst_hbm.at[pl.ds(0, CH)], dis[b],
                              lsems[b]).wait()
        pltpu.make_async_copy(msg_hbm.at[pl.ds(0, CH)], mbs[b],
                              lsems[b]).wait()

    def run(src_ref):
        fire_loads(src_ref, 0, 0)
        fire_loads(src_ref, 1, 1)
        fire_loads(src_ref, 2, 2)

        def body(t, carry):
            for i in range(3):
                j = 3 * t + i

                @pl.when(j < NCHS)
                def _():
                    wait_loads(i)
                    pltpu.sync_copy(mbs[i], acc_s.at[dis[i]], add=True)

                    @pl.when(j + 3 < NCHS)
                    def _():
                        fire_loads(src_ref, j + 3, i)

            return carry

        lax.fori_loop(0, (NCHS + 2) // 3, body, 0)

    @pl.when(cid == 0)
    def _():
        run(msg_hbm)

    @pl.when(cid == 1)
    def _():
        run(exb_hbm)

    plsc.subcore_barrier()

    @pl.when(cid == 0)
    def _():
        pltpu.sync_copy(acc_s.at[pl.ds(row0, TPN)],
                        acc_hbm.at[pl.ds(row0, TPN)])

    @pl.when(cid == 1)
    def _():
        pltpu.sync_copy(acc_s.at[pl.ds(row0, TPN)],
                        den_hbm.at[pl.ds(row0, TPN)])


def kernel(x, edge_index, params):
    src = edge_index[0]
    dst = edge_index[1]
    s2 = jnp.asarray(_S2)

    h = x
    skip = None
    acc = den = None
    for l in range(4):
        wcat = jnp.concatenate(
            [params['l%d_W%s' % (l, nm)] for nm in 'qkvs'], axis=1)
        bcat = jnp.concatenate(
            [params['l%d_b%s' % (l, nm)] for nm in 'qkvs'])[None, :]
        if l == 0:
            q, k, v, s, g = _proj0(h, wcat, bcat, s2)
        else:
            q, k, v, s, g = _merge_proj(acc, den, skip, wcat, bcat, s2)
        pg, vg = _sc_gather(src, dst, q, k, v)
        msg, exb = _e2(pg, vg, g, s2)
        acc, den = _sc_scatter(dst, msg, exb)
        skip = s
    return _final(acc, den, skip, params['Wf'], params['bf'][None, :])
